# Initial kernel scaffold; baseline (speedup 1.0000x reference)
import functools

import jax
import jax.numpy as jnp
from jax import lax
from jax.experimental import pallas as pl
from jax.experimental.pallas import tpu as pltpu
from jax.experimental.pallas import tpu_sc as plsc

N_NODES = 50000
HIDDEN = 256
NUM_GRAPHS = 64
NW = 32
R_PER_W = 1568
CHUNK = 128
NCHUNKS = 13
LANES = 16
NVJ = HIDDEN // LANES
IDS_PAD = 8
NEG_INF = float("-inf")


def _sc_segment_max(z, batch_i32):
  mesh = plsc.VectorSubcoreMesh(core_axis_name="c", subcore_axis_name="s")

  @functools.partial(
      pl.kernel,
      mesh=mesh,
      out_type=jax.ShapeDtypeStruct((NW, NUM_GRAPHS, HIDDEN), jnp.float32),
      scratch_types=[
          pltpu.VMEM((CHUNK, HIDDEN), jnp.float32),
          pltpu.VMEM((IDS_PAD + R_PER_W + 8,), jnp.int32),
          pltpu.VMEM((NUM_GRAPHS, HIDDEN), jnp.float32),
          pltpu.VMEM((NUM_GRAPHS,), jnp.int32),
          pltpu.VMEM((NUM_GRAPHS,), jnp.int32),
      ],
  )
  def body(z_hbm, batch_hbm, out_hbm, buf, ids, acc, bstart, bend):
    wid = lax.axis_index("c") * 16 + lax.axis_index("s")
    rbase = wid * R_PER_W
    rcount = jnp.minimum(R_PER_W, N_NODES - rbase)
    ids_base = jnp.minimum(rbase, N_NODES - R_PER_W)
    off0 = rbase - ids_base

    neg = jnp.full((LANES,), NEG_INF, dtype=jnp.float32)
    zero16 = jnp.zeros((LANES,), dtype=jnp.int32)

    def init_body(k, _):
      acc[k // NVJ, pl.ds((k % NVJ) * LANES, LANES)] = neg
      return 0
    lax.fori_loop(0, NUM_GRAPHS * NVJ, init_body, 0)

    for q in range(NUM_GRAPHS // LANES):
      bstart[pl.ds(q * LANES, LANES)] = zero16
      bend[pl.ds(q * LANES, LANES)] = zero16

    pltpu.sync_copy(batch_hbm.at[pl.ds(ids_base, R_PER_W)],
                    ids.at[pl.ds(IDS_PAD, R_PER_W)])
    ids[off0 + IDS_PAD - 1] = -1
    ids[IDS_PAD + R_PER_W] = -2

    iota = lax.iota(jnp.int32, LANES)

    def scan_body(g, _):
      a0 = off0 + IDS_PAD + g * LANES
      cur = ids[pl.ds(a0, LANES)]
      prev = ids[pl.ds(a0 - 1, LANES)]
      nxt = ids[pl.ds(a0 + 1, LANES)]
      val = g * LANES + iota
      plsc.store_scatter(bstart, [cur], val, mask=cur != prev)
      plsc.store_scatter(bend, [cur], val + 1, mask=cur != nxt)
      return 0
    lax.fori_loop(0, rcount // LANES, scan_body, 0)

    def chunk_body(c, _):
      lb = jnp.minimum(c * CHUNK, rcount - CHUNK)
      pltpu.sync_copy(z_hbm.at[pl.ds(rbase + lb, CHUNK), :], buf)
      s_lo = ids[off0 + IDS_PAD + lb]
      s_hi = ids[off0 + IDS_PAD + lb + CHUNK - 1]

      def seg_body(s, _):
        lo = jnp.maximum(bstart[s], lb)
        hi = jnp.minimum(bend[s], lb + CHUNK)
        hi = jnp.maximum(hi, lo)
        a = tuple(acc[s, pl.ds(j * LANES, LANES)] for j in range(NVJ))

        def row_body(i, a):
          r = i - lb
          return tuple(
              jnp.maximum(a[j], buf[r, pl.ds(j * LANES, LANES)])
              for j in range(NVJ))
        a = lax.fori_loop(lo, hi, row_body, a)
        for j in range(NVJ):
          acc[s, pl.ds(j * LANES, LANES)] = a[j]
        return 0
      lax.fori_loop(s_lo, s_hi + 1, seg_body, 0)
      return 0
    lax.fori_loop(0, NCHUNKS, chunk_body, 0)

    pltpu.sync_copy(acc, out_hbm.at[wid])

  return body(z, batch_i32)


def _tc_merge_head(partials, W, b2):
  def body(p_ref, w_ref, b_ref, o_ref):
    pooled = jnp.max(p_ref[...], axis=0)
    o_ref[...] = lax.dot_general(
        pooled, w_ref[...], (((1,), (1,)), ((), ())),
        preferred_element_type=jnp.float32) + b_ref[...]

  return pl.pallas_call(
      body,
      out_shape=jax.ShapeDtypeStruct((NUM_GRAPHS, W.shape[0]), jnp.float32),
  )(partials, W, b2)


def kernel(z, edge_index, batch, W, b):
  del edge_index
  batch_i32 = batch.astype(jnp.int32)
  partials = _sc_segment_max(z, batch_i32)
  return _tc_merge_head(partials, W, b.reshape(1, -1).astype(jnp.float32))


# speedup vs baseline: 6.5156x; 6.5156x over previous
import functools

import jax
import jax.numpy as jnp
from jax import lax
from jax.experimental import pallas as pl
from jax.experimental.pallas import tpu as pltpu
from jax.experimental.pallas import tpu_sc as plsc

N_NODES = 50000
HIDDEN = 256
NUM_GRAPHS = 64
NW = 32
R_PER_W = 1568
CHUNK = 128
NCHUNKS = 13
LANES = 16
NVJ = HIDDEN // LANES
IDS_PAD = 8
NEG_INF = float("-inf")


def _sc_segment_max(z, batch_i32):
  mesh = plsc.VectorSubcoreMesh(core_axis_name="c", subcore_axis_name="s")

  @functools.partial(
      pl.kernel,
      mesh=mesh,
      compiler_params=pltpu.CompilerParams(needs_layout_passes=False),
      out_type=jax.ShapeDtypeStruct((NW, NUM_GRAPHS, HIDDEN), jnp.float32),
      scratch_types=[
          pltpu.VMEM((CHUNK, HIDDEN), jnp.float32),
          pltpu.VMEM((IDS_PAD + R_PER_W + 24,), jnp.int32),
          pltpu.VMEM((NUM_GRAPHS, HIDDEN), jnp.float32),
          pltpu.VMEM((NUM_GRAPHS + LANES,), jnp.int32),
          pltpu.VMEM((NUM_GRAPHS + LANES,), jnp.int32),
      ],
  )
  def body(z_hbm, batch_hbm, out_hbm, buf, ids, acc, bstart, bend):
    def sget(ref, idx):
      return ref[pl.ds(idx, LANES)][0]

    wid = lax.axis_index("c") * jnp.int32(16) + lax.axis_index("s")
    rbase = wid * R_PER_W
    rcount = jnp.minimum(R_PER_W, N_NODES - rbase)
    ids_base = jnp.minimum(rbase, N_NODES - R_PER_W)
    off0 = rbase - ids_base

    neg = jnp.full((LANES,), NEG_INF, dtype=jnp.float32)
    zero16 = jnp.zeros((LANES,), dtype=jnp.int32)

    def init_body(k, _):
      for j in range(NVJ):
        acc[k, pl.ds(j * LANES, LANES)] = neg
      return 0
    lax.fori_loop(jnp.int32(0), jnp.int32(NUM_GRAPHS), init_body, 0)

    for q in range(NUM_GRAPHS // LANES):
      bstart[pl.ds(q * LANES, LANES)] = zero16
      bend[pl.ds(q * LANES, LANES)] = zero16

    pltpu.sync_copy(batch_hbm.at[pl.ds(ids_base, R_PER_W)],
                    ids.at[pl.ds(IDS_PAD, R_PER_W)])
    iota = lax.iota(jnp.int32, LANES)
    sent_idx = jnp.where(iota == 0, off0 + IDS_PAD - 1,
                         jnp.int32(IDS_PAD + R_PER_W))
    sent_val = jnp.where(iota == 0, jnp.int32(-1), jnp.int32(-2))
    plsc.store_scatter(ids, [sent_idx], sent_val, mask=iota < 2)


    def scan_body(g, _):
      a0 = off0 + IDS_PAD + g * LANES
      cur = ids[pl.ds(a0, LANES)]
      prev = ids[pl.ds(a0 - 1, LANES)]
      nxt = ids[pl.ds(a0 + 1, LANES)]
      val = g * LANES + iota
      plsc.store_scatter(bstart, [cur], val, mask=cur != prev)
      plsc.store_scatter(bend, [cur], val + 1, mask=cur != nxt)
      return 0
    lax.fori_loop(jnp.int32(0), lax.div(rcount, jnp.int32(LANES)), scan_body, 0)

    def chunk_body(c, _):
      lb = jnp.minimum(c * CHUNK, rcount - CHUNK)
      pltpu.sync_copy(z_hbm.at[pl.ds(rbase + lb, CHUNK), :], buf)
      s_lo = sget(ids, off0 + IDS_PAD + lb)
      s_hi = sget(ids, off0 + IDS_PAD + lb + CHUNK - 1)

      def seg_body(s, _):
        lo = jnp.maximum(sget(bstart, s), lb)
        hi = jnp.minimum(sget(bend, s), lb + CHUNK)
        hi = jnp.maximum(hi, lo)
        a = tuple(acc[s, pl.ds(j * LANES, LANES)] for j in range(NVJ))

        def row_body(i, a):
          r = i - lb
          return tuple(
              jnp.maximum(a[j], buf[r, pl.ds(j * LANES, LANES)])
              for j in range(NVJ))
        a = lax.fori_loop(lo, hi, row_body, a)
        for j in range(NVJ):
          acc[s, pl.ds(j * LANES, LANES)] = a[j]
        return 0
      lax.fori_loop(s_lo, s_hi + 1, seg_body, 0)
      return 0
    lax.fori_loop(jnp.int32(0), jnp.int32(NCHUNKS), chunk_body, 0)

    pltpu.sync_copy(acc, out_hbm.at[wid])

  return body(z, batch_i32)


def _tc_merge_head(partials, W, b2):
  def body(p_ref, w_ref, b_ref, o_ref):
    pooled = jnp.max(p_ref[...], axis=0)
    o_ref[...] = lax.dot_general(
        pooled, w_ref[...], (((1,), (1,)), ((), ())),
        preferred_element_type=jnp.float32) + b_ref[...]

  return pl.pallas_call(
      body,
      out_shape=jax.ShapeDtypeStruct((NUM_GRAPHS, W.shape[0]), jnp.float32),
  )(partials, W, b2)


def kernel(z, edge_index, batch, W, b):
  del edge_index
  batch_i32 = batch.astype(jnp.int32)
  partials = _sc_segment_max(z, batch_i32)
  return _tc_merge_head(partials, W, b.reshape(1, -1).astype(jnp.float32))


# speedup vs baseline: 7.9951x; 1.2271x over previous
import functools

import jax
import jax.numpy as jnp
from jax import lax
from jax.experimental import pallas as pl
from jax.experimental.pallas import tpu as pltpu
from jax.experimental.pallas import tpu_sc as plsc

N_NODES = 50000
HIDDEN = 256
NUM_GRAPHS = 64
NW = 32
R_PER_W = 1568
CHUNK = 128
NCHUNKS = 13
LANES = 16
NVJ = HIDDEN // LANES
IDS_PAD = 8
NEG_INF = float("-inf")


def _sc_segment_max(z, batch_i32):
  mesh = plsc.VectorSubcoreMesh(core_axis_name="c", subcore_axis_name="s")

  @functools.partial(
      pl.kernel,
      mesh=mesh,
      compiler_params=pltpu.CompilerParams(needs_layout_passes=False),
      out_type=jax.ShapeDtypeStruct((NW, NUM_GRAPHS, HIDDEN), jnp.float32),
      scratch_types=[
          pltpu.VMEM((CHUNK, HIDDEN), jnp.float32),
          pltpu.VMEM((CHUNK, HIDDEN), jnp.float32),
          pltpu.VMEM((IDS_PAD + R_PER_W + 24,), jnp.int32),
          pltpu.VMEM((NUM_GRAPHS, HIDDEN), jnp.float32),
          pltpu.VMEM((NUM_GRAPHS + LANES,), jnp.int32),
          pltpu.VMEM((NUM_GRAPHS + LANES,), jnp.int32),
          pltpu.SemaphoreType.DMA,
          pltpu.SemaphoreType.DMA,
          pltpu.SemaphoreType.DMA,
      ],
  )
  def body(z_hbm, batch_hbm, out_hbm, buf0, buf1, ids, acc, bstart, bend,
           sem0, sem1, sem_ids):
    def sget(ref, idx):
      return ref[pl.ds(idx, LANES)][0]

    wid = lax.axis_index("c") * jnp.int32(16) + lax.axis_index("s")
    rbase = wid * R_PER_W
    rcount = jnp.minimum(R_PER_W, N_NODES - rbase)
    ids_base = jnp.minimum(rbase, N_NODES - R_PER_W)
    off0 = rbase - ids_base

    ids_h = pltpu.async_copy(batch_hbm.at[pl.ds(ids_base, R_PER_W)],
                             ids.at[pl.ds(IDS_PAD, R_PER_W)], sem_ids)
    neg = jnp.full((LANES,), NEG_INF, dtype=jnp.float32)
    zero16 = jnp.zeros((LANES,), dtype=jnp.int32)

    def init_body(k, _):
      for j in range(NVJ):
        acc[k, pl.ds(j * LANES, LANES)] = neg
      return 0
    lax.fori_loop(jnp.int32(0), jnp.int32(NUM_GRAPHS), init_body, 0)

    for q in range(NUM_GRAPHS // LANES):
      bstart[pl.ds(q * LANES, LANES)] = zero16
      bend[pl.ds(q * LANES, LANES)] = zero16

    ids_h.wait()
    iota = lax.iota(jnp.int32, LANES)
    sent_idx = jnp.where(iota == 0, off0 + IDS_PAD - 1,
                         jnp.int32(IDS_PAD + R_PER_W))
    sent_val = jnp.where(iota == 0, jnp.int32(-1), jnp.int32(-2))
    plsc.store_scatter(ids, [sent_idx], sent_val, mask=iota < 2)


    def scan_body(g, _):
      a0 = off0 + IDS_PAD + g * LANES
      cur = ids[pl.ds(a0, LANES)]
      prev = ids[pl.ds(a0 - 1, LANES)]
      nxt = ids[pl.ds(a0 + 1, LANES)]
      val = g * LANES + iota
      plsc.store_scatter(bstart, [cur], val, mask=cur != prev)
      plsc.store_scatter(bend, [cur], val + 1, mask=cur != nxt)
      return 0
    lax.fori_loop(jnp.int32(0), lax.div(rcount, jnp.int32(LANES)), scan_body, 0)

    def chunk_lb(c):
      return jnp.minimum(jnp.int32(c * CHUNK), rcount - CHUNK)

    def process(buf, lb):
      s_lo = sget(ids, off0 + IDS_PAD + lb)
      s_hi = sget(ids, off0 + IDS_PAD + lb + CHUNK - 1)

      def seg_body(s, _):
        lo = jnp.maximum(sget(bstart, s), lb)
        hi = jnp.minimum(sget(bend, s), lb + CHUNK)
        hi = jnp.maximum(hi, lo)
        a = tuple(acc[s, pl.ds(j * LANES, LANES)] for j in range(NVJ))

        def row_body(i, a):
          r = i - lb
          return tuple(
              jnp.maximum(a[j], buf[r, pl.ds(j * LANES, LANES)])
              for j in range(NVJ))
        a = lax.fori_loop(lo, hi, row_body, a)
        for j in range(NVJ):
          acc[s, pl.ds(j * LANES, LANES)] = a[j]
        return 0
      lax.fori_loop(s_lo, s_hi + 1, seg_body, 0)

    bufs, sems = (buf0, buf1), (sem0, sem1)
    handles = [None, None]
    handles[0] = pltpu.async_copy(
        z_hbm.at[pl.ds(rbase + chunk_lb(0), CHUNK), :], buf0, sem0)
    for c in range(NCHUNKS):
      b = c % 2
      handles[b].wait()
      if c + 1 < NCHUNKS:
        nb = (c + 1) % 2
        handles[nb] = pltpu.async_copy(
            z_hbm.at[pl.ds(rbase + chunk_lb(c + 1), CHUNK), :],
            bufs[nb], sems[nb])
      process(bufs[b], chunk_lb(c))

    pltpu.sync_copy(acc, out_hbm.at[wid])

  return body(z, batch_i32)


def _tc_merge_head(partials, W, b2):
  def body(p_ref, w_ref, b_ref, o_ref):
    pooled = jnp.max(p_ref[...], axis=0)
    o_ref[...] = lax.dot_general(
        pooled, w_ref[...], (((1,), (1,)), ((), ())),
        preferred_element_type=jnp.float32) + b_ref[...]

  return pl.pallas_call(
      body,
      out_shape=jax.ShapeDtypeStruct((NUM_GRAPHS, W.shape[0]), jnp.float32),
  )(partials, W, b2)


def kernel(z, edge_index, batch, W, b):
  del edge_index
  batch_i32 = batch.astype(jnp.int32)
  partials = _sc_segment_max(z, batch_i32)
  return _tc_merge_head(partials, W, b.reshape(1, -1).astype(jnp.float32))
